# Initial kernel scaffold; baseline (speedup 1.0000x reference)
import jax
import jax.numpy as jnp
from jax.experimental import pallas as pl


def kernel(native_x, x, edge_index, batch, y, emb, W_aa, b_aa, W_esm, b_esm, Wg0, bg0, Wg1, bg1, Wg2, bg2, Wl1, bl1, Wl2, bl2, Wr1, br1, Wr2, br2):
    raise NotImplementedError("write your pallas kernel here")


# speedup vs baseline: 3.3953x; 3.3953x over previous
import functools

import jax
import jax.numpy as jnp
from jax import lax
from jax.experimental import pallas as pl
from jax.experimental.pallas import tpu as pltpu
from jax.experimental.pallas import tpu_sc as plsc

f32 = jnp.float32

NN = 10000
NP = 10240
EE = 160000
EPAD = 163840
NTILES = 32
CHUNKS = 40
CW = 128
ROWS_PT = NP // 16
GG = 16
MB = 1024
NBLK = NP // MB
FD = 512
NSLAB = 4
SW = FD // NSLAB


@functools.lru_cache(maxsize=None)
def _sc_kernels():
    mesh = plsc.VectorSubcoreMesh(core_axis_name="c", subcore_axis_name="s")

    @functools.partial(
        pl.kernel, mesh=mesh,
        out_type=jax.ShapeDtypeStruct((2, NP, SW), f32),
        scratch_types=[
            pltpu.VMEM((CHUNKS, CW), jnp.int32),
            pltpu.VMEM((CW, SW), f32),
            pltpu.VMEM_SHARED((NP, SW), f32),
        ],
    )
    def deg_kernel(ec_hbm, ones_hbm, z_hbm, out_hbm, ci_v, ones_v, acc_sh):
        c = lax.axis_index("c")
        s = lax.axis_index("s")
        w = c * 16 + s
        pltpu.sync_copy(ec_hbm.at[w], ci_v)
        pltpu.sync_copy(ones_hbm, ones_v)

        @pl.when(s == 0)
        def _():
            pltpu.sync_copy(z_hbm, acc_sh)

        plsc.subcore_barrier()

        @pl.loop(0, CHUNKS)
        def _(j):
            pltpu.sync_copy(ones_v, acc_sh.at[ci_v.at[j]], add=True)

        plsc.subcore_barrier()

        @pl.when(s == 0)
        def _():
            pltpu.sync_copy(acc_sh, out_hbm.at[c])

    @functools.partial(
        pl.kernel, mesh=mesh,
        out_type=jax.ShapeDtypeStruct((2, NSLAB, NP, SW), f32),
        scratch_types=[
            pltpu.VMEM((CHUNKS, CW), jnp.int32),
            pltpu.VMEM((CHUNKS, CW), jnp.int32),
            pltpu.VMEM((CW, SW), f32),
            pltpu.VMEM_SHARED((NP, SW), f32),
        ],
    )
    def mp_kernel(u0, u1, u2, u3, er_hbm, ec_hbm, z_hbm, out_hbm,
                  ri_v, ci_v, g_v, acc_sh):
        c = lax.axis_index("c")
        s = lax.axis_index("s")
        w = c * 16 + s
        pltpu.sync_copy(er_hbm.at[w], ri_v)
        pltpu.sync_copy(ec_hbm.at[w], ci_v)
        tabs = [u0, u1, u2, u3]
        for slab in range(NSLAB):
            @pl.when(s == 0)
            def _():
                pltpu.sync_copy(z_hbm, acc_sh)

            plsc.subcore_barrier()

            @pl.loop(0, CHUNKS)
            def _(j):
                pltpu.sync_copy(tabs[slab].at[ri_v.at[j]], g_v)
                pltpu.sync_copy(g_v, acc_sh.at[ci_v.at[j]], add=True)

            plsc.subcore_barrier()

            @pl.when(s == 0)
            def _():
                pltpu.sync_copy(acc_sh, out_hbm.at[c].at[slab])

            plsc.subcore_barrier()

    return deg_kernel, mp_kernel


def _k1_body(xb, nx3, degp, emb, waa, wesm, wg0, b2,
             dis_o, u0_o, u1_o, u2_o, u3_o):
    deg = degp[0, :, 0:1] + degp[1, :, 0:1]
    dis = lax.rsqrt(deg + 1.0)
    dis_o[0, 0, :] = dis[:, 0]
    nx = nx3[0, 0, :]
    oh = (nx[:, None] == lax.broadcasted_iota(jnp.int32, (1, 21), 1)
          ).astype(f32)
    t21 = jnp.dot(emb[...], waa[...], preferred_element_type=f32)
    h0 = (oh @ t21
          + jnp.dot(xb[...], wesm[...], preferred_element_type=f32)
          + b2[...])
    h0 = jnp.maximum(h0, 0.0)
    u = jnp.dot(h0, wg0[...], preferred_element_type=f32) * dis
    u0_o[...] = u[:, 0:128]
    u1_o[...] = u[:, 128:256]
    u2_o[...] = u[:, 256:384]
    u3_o[...] = u[:, 384:512]


def _comb_body(has_res, sp, u0, u1, u2, u3, dis3, hprev, wn, b2,
               h_o, u0_o, u1_o, u2_o, u3_o):
    s = sp[0] + sp[1]
    scat = jnp.concatenate([s[0], s[1], s[2], s[3]], axis=1)
    u = jnp.concatenate([u0[...], u1[...], u2[...], u3[...]], axis=1)
    dis = dis3[0, 0, :][:, None]
    out = dis * (scat + u) + b2[...]
    a = jnp.maximum(out, 0.0)
    h = hprev[...] + a if has_res else a
    h_o[...] = h
    un = jnp.dot(h, wn[...], preferred_element_type=f32) * dis
    u0_o[...] = un[:, 0:128]
    u1_o[...] = un[:, 128:256]
    u2_o[...] = un[:, 256:384]
    u3_o[...] = un[:, 384:512]


def _comb2_body(sp, u0, u1, u2, u3, dis3, hprev, batm, b2, gm_o):
    i = pl.program_id(0)
    s = sp[0] + sp[1]
    scat = jnp.concatenate([s[0], s[1], s[2], s[3]], axis=1)
    u = jnp.concatenate([u0[...], u1[...], u2[...], u3[...]], axis=1)
    dis = dis3[0, 0, :][:, None]
    out = dis * (scat + u) + b2[...]
    h = hprev[...] + jnp.maximum(out, 0.0)

    @pl.when(i == 0)
    def _():
        gm_o[...] = jnp.full((GG, FD), -jnp.inf, f32)

    for g in range(GG):
        m = batm[:, g:g + 1] > 0.0
        vals = jnp.where(m, h, -jnp.inf)
        gm_o[g, :] = jnp.maximum(gm_o[g, :], jnp.max(vals, axis=0))


def _head_body(gm, y, wl1, bl1, wl2, bl2, wr1, br1, wr2, br2, yp_o, lab_o):
    l1 = jnp.maximum(jnp.dot(y[...], wl1[...], preferred_element_type=f32)
                     + bl1[...], 0.0)
    lab_o[...] = jnp.maximum(
        jnp.dot(l1, wl2[...], preferred_element_type=f32) + bl2[...], 0.0)
    r1 = jnp.maximum(jnp.dot(gm[...], wr1[...], preferred_element_type=f32)
                     + br1[...], 0.0)
    yp_o[...] = jax.nn.sigmoid(
        jnp.dot(r1, wr2[...], preferred_element_type=f32) + br2[...])


def _full(shape):
    return pl.BlockSpec(shape, lambda *_: tuple(0 for _ in shape))


_K1 = pl.pallas_call(
    _k1_body,
    grid=(NBLK,),
    in_specs=[
        pl.BlockSpec((MB, 1280), lambda i: (i, 0)),
        pl.BlockSpec((1, 1, MB), lambda i: (i, 0, 0)),
        pl.BlockSpec((2, MB, SW), lambda i: (0, i, 0)),
        _full((21, 96)),
        _full((96, FD)),
        _full((1280, FD)),
        _full((FD, FD)),
        _full((1, FD)),
    ],
    out_specs=[
        pl.BlockSpec((1, 1, MB), lambda i: (i, 0, 0)),
        pl.BlockSpec((MB, SW), lambda i: (i, 0)),
        pl.BlockSpec((MB, SW), lambda i: (i, 0)),
        pl.BlockSpec((MB, SW), lambda i: (i, 0)),
        pl.BlockSpec((MB, SW), lambda i: (i, 0)),
    ],
    out_shape=[
        jax.ShapeDtypeStruct((NBLK, 1, MB), f32),
        jax.ShapeDtypeStruct((NP, SW), f32),
        jax.ShapeDtypeStruct((NP, SW), f32),
        jax.ShapeDtypeStruct((NP, SW), f32),
        jax.ShapeDtypeStruct((NP, SW), f32),
    ],
)

_slab_in = pl.BlockSpec((MB, SW), lambda i: (i, 0))
_sp_in = pl.BlockSpec((2, NSLAB, MB, SW), lambda i: (0, 0, i, 0))
_dis_in = pl.BlockSpec((1, 1, MB), lambda i: (i, 0, 0))
_h_in = pl.BlockSpec((MB, FD), lambda i: (i, 0))

_slab_outs = [pl.BlockSpec((MB, SW), lambda i: (i, 0)) for _ in range(NSLAB)]
_slab_shapes = [jax.ShapeDtypeStruct((NP, SW), f32) for _ in range(NSLAB)]


def _make_comb(has_res):
    ins = [_sp_in, _slab_in, _slab_in, _slab_in, _slab_in, _dis_in]
    if has_res:
        ins.append(_h_in)
    ins += [_full((FD, FD)), _full((1, FD))]
    if has_res:
        body = functools.partial(_comb_body, True)
    else:
        def body(sp, u0, u1, u2, u3, dis3, wn, b2, *outs):
            _comb_body(False, sp, u0, u1, u2, u3, dis3, None, wn, b2, *outs)
    return pl.pallas_call(
        body,
        grid=(NBLK,),
        in_specs=ins,
        out_specs=[_h_in] + _slab_outs,
        out_shape=[jax.ShapeDtypeStruct((NP, FD), f32)] + _slab_shapes,
    )


_COMB0 = _make_comb(False)
_COMB1 = _make_comb(True)

_COMB2 = pl.pallas_call(
    _comb2_body,
    grid=(NBLK,),
    in_specs=[_sp_in, _slab_in, _slab_in, _slab_in, _slab_in, _dis_in,
              _h_in, pl.BlockSpec((MB, GG), lambda i: (i, 0)),
              _full((1, FD))],
    out_specs=pl.BlockSpec((GG, FD), lambda i: (0, 0)),
    out_shape=jax.ShapeDtypeStruct((GG, FD), f32),
)

_HEAD = pl.pallas_call(
    _head_body,
    in_specs=[_full((GG, FD)), _full((GG, 256)),
              _full((256, 1024)), _full((1, 1024)),
              _full((1024, FD)), _full((1, FD)),
              _full((FD, 1024)), _full((1, 1024)),
              _full((1024, 256)), _full((1, 256))],
    out_specs=[_full((GG, 256)), _full((GG, FD))],
    out_shape=[jax.ShapeDtypeStruct((GG, 256), f32),
               jax.ShapeDtypeStruct((GG, FD), f32)],
)


def kernel(native_x, x, edge_index, batch, y, emb, W_aa, b_aa, W_esm, b_esm,
           Wg0, bg0, Wg1, bg1, Wg2, bg2, Wl1, bl1, Wl2, bl2,
           Wr1, br1, Wr2, br2):
    i32 = jnp.int32
    er = jnp.concatenate(
        [edge_index[0].astype(i32), jnp.zeros((EPAD - EE,), i32)]
    ).reshape(NTILES, CHUNKS, CW)
    ec = jnp.concatenate(
        [edge_index[1].astype(i32), jnp.full((EPAD - EE,), NN, i32)]
    ).reshape(NTILES, CHUNKS, CW)
    x_p = jnp.pad(x, ((0, NP - NN), (0, 0)))
    nx3 = jnp.pad(native_x.astype(i32), (0, NP - NN)).reshape(NBLK, 1, MB)
    bat_p = jnp.pad(batch.astype(i32), (0, NP - NN), constant_values=GG)
    batm = (bat_p[:, None] == jnp.arange(GG, dtype=i32)[None, :]).astype(f32)
    ones_c = jnp.ones((CW, SW), f32)
    zz = jnp.zeros((NP, SW), f32)

    _deg_kernel, _mp_kernel = _sc_kernels()
    degp = _deg_kernel(ec, ones_c, zz)
    b01 = (b_aa + b_esm).reshape(1, FD)
    dis3, u0, u1, u2, u3 = _K1(x_p, nx3, degp, emb, W_aa, W_esm, Wg0, b01)

    sp0 = _mp_kernel(u0, u1, u2, u3, er, ec, zz)
    h1, v0, v1, v2, v3 = _COMB0(sp0, u0, u1, u2, u3, dis3,
                                Wg1, bg0.reshape(1, FD))
    sp1 = _mp_kernel(v0, v1, v2, v3, er, ec, zz)
    h2, w0, w1, w2, w3 = _COMB1(sp1, v0, v1, v2, v3, dis3, h1,
                                Wg2, bg1.reshape(1, FD))
    sp2 = _mp_kernel(w0, w1, w2, w3, er, ec, zz)
    gmax = _COMB2(sp2, w0, w1, w2, w3, dis3, h2, batm, bg2.reshape(1, FD))

    y_pred, lab = _HEAD(gmax, y, Wl1, bl1.reshape(1, 1024),
                        Wl2, bl2.reshape(1, FD),
                        Wr1, br1.reshape(1, 1024), Wr2, br2.reshape(1, 256))
    return (y_pred, gmax, lab)


# speedup vs baseline: 3.5933x; 1.0583x over previous
import functools

import jax
import jax.numpy as jnp
from jax import lax
from jax.experimental import pallas as pl
from jax.experimental.pallas import tpu as pltpu
from jax.experimental.pallas import tpu_sc as plsc

f32 = jnp.float32

NN = 10000
NP = 10240
EE = 160000
EPAD = 163840
NTILES = 32
CHUNKS = 40
CW = 128
ROWS_PT = NP // 16
GG = 16
MB = 1024
NBLK = NP // MB
FD = 512
NSLAB = 4
SW = FD // NSLAB


@functools.lru_cache(maxsize=None)
def _sc_kernels():
    mesh = plsc.VectorSubcoreMesh(core_axis_name="c", subcore_axis_name="s")

    @functools.partial(
        pl.kernel, mesh=mesh,
        out_type=jax.ShapeDtypeStruct((2, NP, SW), f32),
        scratch_types=[
            pltpu.VMEM((CHUNKS, CW), jnp.int32),
            pltpu.VMEM((CW, SW), f32),
            pltpu.VMEM_SHARED((NP, SW), f32),
        ],
    )
    def deg_kernel(ec_hbm, ones_hbm, z_hbm, out_hbm, ci_v, ones_v, acc_sh):
        c = lax.axis_index("c")
        s = lax.axis_index("s")
        w = c * 16 + s
        pltpu.sync_copy(ec_hbm.at[w], ci_v)
        pltpu.sync_copy(ones_hbm, ones_v)

        @pl.when(s == 0)
        def _():
            pltpu.sync_copy(z_hbm, acc_sh)

        plsc.subcore_barrier()

        @pl.loop(0, CHUNKS)
        def _(j):
            pltpu.sync_copy(ones_v, acc_sh.at[ci_v.at[j]], add=True)

        plsc.subcore_barrier()

        @pl.when(s == 0)
        def _():
            pltpu.sync_copy(acc_sh, out_hbm.at[c])

    @functools.partial(
        pl.kernel, mesh=mesh,
        out_type=jax.ShapeDtypeStruct((2, NSLAB, NP, SW), f32),
        scratch_types=[
            pltpu.VMEM((CHUNKS, CW), jnp.int32),
            pltpu.VMEM((CHUNKS, CW), jnp.int32),
            pltpu.VMEM((CW, SW), f32),
            pltpu.VMEM((CW, SW), f32),
            pltpu.SemaphoreType.DMA,
            pltpu.SemaphoreType.DMA,
            pltpu.SemaphoreType.DMA,
            pltpu.VMEM_SHARED((NP, SW), f32),
        ],
    )
    def mp_kernel(u0, u1, u2, u3, er_hbm, ec_hbm, z_hbm, out_hbm,
                  ri_v, ci_v, ga_v, gb_v, sem_a, sem_b, sem_s, acc_sh):
        c = lax.axis_index("c")
        s = lax.axis_index("s")
        w = c * 16 + s
        pltpu.sync_copy(er_hbm.at[w], ri_v)
        pltpu.sync_copy(ec_hbm.at[w], ci_v)
        tabs = [u0, u1, u2, u3]
        for slab in range(NSLAB):
            tab = tabs[slab]

            @pl.when(s == 0)
            def _():
                pltpu.sync_copy(z_hbm, acc_sh)

            plsc.subcore_barrier()

            pltpu.make_async_copy(tab.at[ri_v.at[0]], ga_v, sem_a).start()

            @pl.loop(0, CHUNKS // 2)
            def _(jj):
                j0 = jj * 2
                j1 = j0 + 1
                pltpu.make_async_copy(tab.at[ri_v.at[j0]], ga_v, sem_a).wait()
                pltpu.make_async_copy(tab.at[ri_v.at[j1]], gb_v, sem_b).start()
                sa = pltpu.make_async_copy(ga_v, acc_sh.at[ci_v.at[j0]],
                                           sem_s)
                sa.start(add=True)
                sa.wait()
                pltpu.make_async_copy(tab.at[ri_v.at[j1]], gb_v, sem_b).wait()
                nxt = jnp.minimum(j0 + 2, CHUNKS - 1)
                pltpu.make_async_copy(tab.at[ri_v.at[nxt]], ga_v,
                                      sem_a).start()
                sb = pltpu.make_async_copy(gb_v, acc_sh.at[ci_v.at[j1]],
                                           sem_s)
                sb.start(add=True)
                sb.wait()

            pltpu.make_async_copy(tab.at[ri_v.at[0]], ga_v, sem_a).wait()

            plsc.subcore_barrier()

            @pl.when(s == 0)
            def _():
                pltpu.sync_copy(acc_sh, out_hbm.at[c].at[slab])

            plsc.subcore_barrier()

    return deg_kernel, mp_kernel


def _k1_body(xb, nx3, degp, emb, waa, wesm, wg0, b2,
             dis_o, u0_o, u1_o, u2_o, u3_o):
    deg = degp[0, :, 0:1] + degp[1, :, 0:1]
    dis = lax.rsqrt(deg + 1.0)
    dis_o[0, 0, :] = dis[:, 0]
    nx = nx3[0, 0, :]
    oh = (nx[:, None] == lax.broadcasted_iota(jnp.int32, (1, 21), 1)
          ).astype(f32)
    t21 = jnp.dot(emb[...], waa[...], preferred_element_type=f32)
    h0 = (oh @ t21
          + jnp.dot(xb[...], wesm[...], preferred_element_type=f32)
          + b2[...])
    h0 = jnp.maximum(h0, 0.0)
    u = jnp.dot(h0, wg0[...], preferred_element_type=f32) * dis
    u0_o[...] = u[:, 0:128]
    u1_o[...] = u[:, 128:256]
    u2_o[...] = u[:, 256:384]
    u3_o[...] = u[:, 384:512]


def _comb_body(has_res, sp, u0, u1, u2, u3, dis3, hprev, wn, b2,
               h_o, u0_o, u1_o, u2_o, u3_o):
    s = sp[0] + sp[1]
    scat = jnp.concatenate([s[0], s[1], s[2], s[3]], axis=1)
    u = jnp.concatenate([u0[...], u1[...], u2[...], u3[...]], axis=1)
    dis = dis3[0, 0, :][:, None]
    out = dis * (scat + u) + b2[...]
    a = jnp.maximum(out, 0.0)
    h = hprev[...] + a if has_res else a
    h_o[...] = h
    un = jnp.dot(h, wn[...], preferred_element_type=f32) * dis
    u0_o[...] = un[:, 0:128]
    u1_o[...] = un[:, 128:256]
    u2_o[...] = un[:, 256:384]
    u3_o[...] = un[:, 384:512]


def _comb2_body(sp, u0, u1, u2, u3, dis3, hprev, batm, b2, gm_o):
    i = pl.program_id(0)
    s = sp[0] + sp[1]
    scat = jnp.concatenate([s[0], s[1], s[2], s[3]], axis=1)
    u = jnp.concatenate([u0[...], u1[...], u2[...], u3[...]], axis=1)
    dis = dis3[0, 0, :][:, None]
    out = dis * (scat + u) + b2[...]
    h = hprev[...] + jnp.maximum(out, 0.0)

    @pl.when(i == 0)
    def _():
        gm_o[...] = jnp.full((GG, FD), -jnp.inf, f32)

    for g in range(GG):
        m = batm[:, g:g + 1] > 0.0
        vals = jnp.where(m, h, -jnp.inf)
        gm_o[g, :] = jnp.maximum(gm_o[g, :], jnp.max(vals, axis=0))


def _head_body(gm, y, wl1, bl1, wl2, bl2, wr1, br1, wr2, br2, yp_o, lab_o):
    l1 = jnp.maximum(jnp.dot(y[...], wl1[...], preferred_element_type=f32)
                     + bl1[...], 0.0)
    lab_o[...] = jnp.maximum(
        jnp.dot(l1, wl2[...], preferred_element_type=f32) + bl2[...], 0.0)
    r1 = jnp.maximum(jnp.dot(gm[...], wr1[...], preferred_element_type=f32)
                     + br1[...], 0.0)
    yp_o[...] = jax.nn.sigmoid(
        jnp.dot(r1, wr2[...], preferred_element_type=f32) + br2[...])


def _full(shape):
    return pl.BlockSpec(shape, lambda *_: tuple(0 for _ in shape))


_K1 = pl.pallas_call(
    _k1_body,
    grid=(NBLK,),
    in_specs=[
        pl.BlockSpec((MB, 1280), lambda i: (i, 0)),
        pl.BlockSpec((1, 1, MB), lambda i: (i, 0, 0)),
        pl.BlockSpec((2, MB, SW), lambda i: (0, i, 0)),
        _full((21, 96)),
        _full((96, FD)),
        _full((1280, FD)),
        _full((FD, FD)),
        _full((1, FD)),
    ],
    out_specs=[
        pl.BlockSpec((1, 1, MB), lambda i: (i, 0, 0)),
        pl.BlockSpec((MB, SW), lambda i: (i, 0)),
        pl.BlockSpec((MB, SW), lambda i: (i, 0)),
        pl.BlockSpec((MB, SW), lambda i: (i, 0)),
        pl.BlockSpec((MB, SW), lambda i: (i, 0)),
    ],
    out_shape=[
        jax.ShapeDtypeStruct((NBLK, 1, MB), f32),
        jax.ShapeDtypeStruct((NP, SW), f32),
        jax.ShapeDtypeStruct((NP, SW), f32),
        jax.ShapeDtypeStruct((NP, SW), f32),
        jax.ShapeDtypeStruct((NP, SW), f32),
    ],
)

_slab_in = pl.BlockSpec((MB, SW), lambda i: (i, 0))
_sp_in = pl.BlockSpec((2, NSLAB, MB, SW), lambda i: (0, 0, i, 0))
_dis_in = pl.BlockSpec((1, 1, MB), lambda i: (i, 0, 0))
_h_in = pl.BlockSpec((MB, FD), lambda i: (i, 0))

_slab_outs = [pl.BlockSpec((MB, SW), lambda i: (i, 0)) for _ in range(NSLAB)]
_slab_shapes = [jax.ShapeDtypeStruct((NP, SW), f32) for _ in range(NSLAB)]


def _make_comb(has_res):
    ins = [_sp_in, _slab_in, _slab_in, _slab_in, _slab_in, _dis_in]
    if has_res:
        ins.append(_h_in)
    ins += [_full((FD, FD)), _full((1, FD))]
    if has_res:
        body = functools.partial(_comb_body, True)
    else:
        def body(sp, u0, u1, u2, u3, dis3, wn, b2, *outs):
            _comb_body(False, sp, u0, u1, u2, u3, dis3, None, wn, b2, *outs)
    return pl.pallas_call(
        body,
        grid=(NBLK,),
        in_specs=ins,
        out_specs=[_h_in] + _slab_outs,
        out_shape=[jax.ShapeDtypeStruct((NP, FD), f32)] + _slab_shapes,
    )


_COMB0 = _make_comb(False)
_COMB1 = _make_comb(True)

_COMB2 = pl.pallas_call(
    _comb2_body,
    grid=(NBLK,),
    in_specs=[_sp_in, _slab_in, _slab_in, _slab_in, _slab_in, _dis_in,
              _h_in, pl.BlockSpec((MB, GG), lambda i: (i, 0)),
              _full((1, FD))],
    out_specs=pl.BlockSpec((GG, FD), lambda i: (0, 0)),
    out_shape=jax.ShapeDtypeStruct((GG, FD), f32),
)

_HEAD = pl.pallas_call(
    _head_body,
    in_specs=[_full((GG, FD)), _full((GG, 256)),
              _full((256, 1024)), _full((1, 1024)),
              _full((1024, FD)), _full((1, FD)),
              _full((FD, 1024)), _full((1, 1024)),
              _full((1024, 256)), _full((1, 256))],
    out_specs=[_full((GG, 256)), _full((GG, FD))],
    out_shape=[jax.ShapeDtypeStruct((GG, 256), f32),
               jax.ShapeDtypeStruct((GG, FD), f32)],
)


def kernel(native_x, x, edge_index, batch, y, emb, W_aa, b_aa, W_esm, b_esm,
           Wg0, bg0, Wg1, bg1, Wg2, bg2, Wl1, bl1, Wl2, bl2,
           Wr1, br1, Wr2, br2):
    i32 = jnp.int32
    er = jnp.concatenate(
        [edge_index[0].astype(i32), jnp.zeros((EPAD - EE,), i32)]
    ).reshape(NTILES, CHUNKS, CW)
    ec = jnp.concatenate(
        [edge_index[1].astype(i32), jnp.full((EPAD - EE,), NN, i32)]
    ).reshape(NTILES, CHUNKS, CW)
    x_p = jnp.pad(x, ((0, NP - NN), (0, 0)))
    nx3 = jnp.pad(native_x.astype(i32), (0, NP - NN)).reshape(NBLK, 1, MB)
    bat_p = jnp.pad(batch.astype(i32), (0, NP - NN), constant_values=GG)
    batm = (bat_p[:, None] == jnp.arange(GG, dtype=i32)[None, :]).astype(f32)
    ones_c = jnp.ones((CW, SW), f32)
    zz = jnp.zeros((NP, SW), f32)

    _deg_kernel, _mp_kernel = _sc_kernels()
    degp = _deg_kernel(ec, ones_c, zz)
    b01 = (b_aa + b_esm).reshape(1, FD)
    dis3, u0, u1, u2, u3 = _K1(x_p, nx3, degp, emb, W_aa, W_esm, Wg0, b01)

    sp0 = _mp_kernel(u0, u1, u2, u3, er, ec, zz)
    h1, v0, v1, v2, v3 = _COMB0(sp0, u0, u1, u2, u3, dis3,
                                Wg1, bg0.reshape(1, FD))
    sp1 = _mp_kernel(v0, v1, v2, v3, er, ec, zz)
    h2, w0, w1, w2, w3 = _COMB1(sp1, v0, v1, v2, v3, dis3, h1,
                                Wg2, bg1.reshape(1, FD))
    sp2 = _mp_kernel(w0, w1, w2, w3, er, ec, zz)
    gmax = _COMB2(sp2, w0, w1, w2, w3, dis3, h2, batm, bg2.reshape(1, FD))

    y_pred, lab = _HEAD(gmax, y, Wl1, bl1.reshape(1, 1024),
                        Wl2, bl2.reshape(1, FD),
                        Wr1, br1.reshape(1, 1024), Wr2, br2.reshape(1, 256))
    return (y_pred, gmax, lab)
